# baseline (speedup 1.0000x reference)
import functools

import jax
import jax.numpy as jnp
from jax import lax
from jax.experimental import pallas as pl
from jax.experimental.pallas import tpu as pltpu
from jax.experimental.pallas import tpu_sc as plsc

N = 10000
E = 320000
D = 128
NC, NS = 2, 16
NW = NC * NS
CH = 128
N_CHUNKS = E // CH
NT_BASE = N_CHUNKS // NW
NT_XTRA = N_CHUNKS % NW
RPT = 624
REM_BASE = NS * RPT
REM = N - REM_BASE


def _edge_body(x_hbm, src_hbm, dst_hbm, a_hbm, web_hbm, out_hbm,
               sb0, sb1, sb2, db0, db1, db2, av0, av1, av2,
               rows0, rows1, rows2,
               web_v, agg_sh, sem_g, sem_s, sem_i, sem_c, sem_d):
    sb_s = (sb0, sb1, sb2)
    db_s = (db0, db1, db2)
    av_s = (av0, av1, av2)
    rows_s = (rows0, rows1, rows2)
    c = lax.axis_index("c")
    s = lax.axis_index("s")
    w = s * NC + c

    def _zrow(i, _):
        for j in range(8):
            rows0[i, pl.ds(j * 16, 16)] = jnp.zeros((16,), jnp.float32)
        return 0
    lax.fori_loop(0, CH, _zrow, 0)
    zsrc = rows0
    base_r = s * RPT
    for kk in range(RPT // CH):
        pltpu.sync_copy(zsrc, agg_sh.at[pl.ds(base_r + kk * CH, CH)])
    rem = RPT % CH
    pltpu.sync_copy(zsrc.at[pl.ds(0, rem)],
                    agg_sh.at[pl.ds(base_r + (RPT // CH) * CH, rem)])

    @pl.when(s == NS - 1)
    def _zero_tail():
        pltpu.sync_copy(zsrc.at[pl.ds(0, REM)],
                        agg_sh.at[pl.ds(REM_BASE, REM)])

    pltpu.sync_copy(web_hbm, web_v)
    plsc.subcore_barrier()

    wej = [web_v[0, pl.ds(j * 16, 16)] for j in range(8)]

    has_extra = w < NT_XTRA

    def _gather(slot):
        return pltpu.make_async_copy(
            x_hbm.at[sb_s[slot]], rows_s[slot], sem_g)

    def _scatter(slot):
        return pltpu.make_async_copy(rows_s[slot],
                                     agg_sh.at[db_s[slot]], sem_s)

    def _scopy(t, slot):
        return pltpu.make_async_copy(src_hbm.at[w + t * NW], sb_s[slot],
                                     sem_c)

    def _dcopy(t, slot):
        return pltpu.make_async_copy(dst_hbm.at[w + t * NW], db_s[slot],
                                     sem_d)

    def _acopy(t, slot):
        return pltpu.make_async_copy(a_hbm.at[w + t * NW], av_s[slot],
                                     sem_i)

    def _compute(t, slot):
        _acopy(t, slot).wait()
        _gather(slot).wait()
        rv = rows_s[slot]
        av = av_s[slot]

        @plsc.parallel_loop(0, CH // 16, unroll=4)
        def _edges(i16):
            a16 = av[pl.ds(i16 * 16, 16)]
            for u in range(16):
                i = i16 * 16 + u
                a = a16[u]
                for jj in range(8):
                    sl = pl.ds(jj * 16, 16)
                    rv[i, sl] = jnp.maximum(
                        rv[i, sl] + a * wej[jj], 0.0)

    def _scatter_start(slot):
        pltpu.async_copy(rows_s[slot], agg_sh.at[db_s[slot]], sem_s,
                         add=True)

    _scopy(0, 0).start()
    _scopy(0, 0).wait()
    _gather(0).start()
    _scopy(1, 1).start()
    _dcopy(0, 0).start()
    _acopy(0, 0).start()
    _acopy(1, 1).start()
    plsc.subcore_barrier()

    def _chunk(t, slot, waitprev_cond, prefetch_cond, acopy_cond):
        @pl.when(waitprev_cond)
        def _wp():
            _scatter((slot + 1) % 3).wait()

        @pl.when(prefetch_cond)
        def _pf():
            _dcopy(t + 1, (slot + 1) % 3).start()
            _scopy(t + 1, (slot + 1) % 3).wait()
            _gather((slot + 1) % 3).start()

        @pl.when(acopy_cond)
        def _ac():
            _scopy(t + 2, (slot + 2) % 3).start()
            _acopy(t + 2, (slot + 2) % 3).start()
        _compute(t, slot)
        _dcopy(t, slot).wait()
        _scatter_start(slot)

    def _trip(p, _):
        t0 = p * 3
        last_trip = p == NT_BASE // 3 - 1
        _chunk(t0, 0, p > 0, True, True)
        _chunk(t0 + 1, 1, p > 0, True, (~last_trip) | has_extra)
        _chunk(t0 + 2, 2, True, (~last_trip) | has_extra, ~last_trip)
        return 0
    lax.fori_loop(0, NT_BASE // 3, _trip, 0)

    _scatter(1).wait()

    @pl.when(has_extra)
    def _extra():
        _compute(NT_BASE, 0)
        _dcopy(NT_BASE, 0).wait()
        _scatter_start(0)

    _scatter(2).wait()

    @pl.when(has_extra)
    def _wait_extra():
        _scatter(0).wait()

    plsc.subcore_barrier()
    pltpu.sync_copy(agg_sh.at[pl.ds(s * RPT, RPT)],
                    out_hbm.at[c, pl.ds(s * RPT, RPT)])

    @pl.when(s == NS - 1)
    def _dump_tail():
        pltpu.sync_copy(agg_sh.at[pl.ds(REM_BASE, REM)],
                        out_hbm.at[c, pl.ds(REM_BASE, REM)])


@functools.cache
def _edge_call():
    return pl.kernel(
        _edge_body,
        out_type=jax.ShapeDtypeStruct((NC, N, D), jnp.float32),
        mesh=plsc.VectorSubcoreMesh(core_axis_name="c", subcore_axis_name="s",
                                    num_cores=NC, num_subcores=NS),
        scratch_types=[
            pltpu.VMEM((CH,), jnp.int32),
            pltpu.VMEM((CH,), jnp.int32),
            pltpu.VMEM((CH,), jnp.int32),
            pltpu.VMEM((CH,), jnp.int32),
            pltpu.VMEM((CH,), jnp.int32),
            pltpu.VMEM((CH,), jnp.int32),
            pltpu.VMEM((CH,), jnp.float32),
            pltpu.VMEM((CH,), jnp.float32),
            pltpu.VMEM((CH,), jnp.float32),
            pltpu.VMEM((CH, D), jnp.float32),
            pltpu.VMEM((CH, D), jnp.float32),
            pltpu.VMEM((CH, D), jnp.float32),
            pltpu.VMEM((1, D), jnp.float32),
            pltpu.VMEM_SHARED((N, D), jnp.float32),
            pltpu.SemaphoreType.DMA,
            pltpu.SemaphoreType.DMA,
            pltpu.SemaphoreType.DMA,
            pltpu.SemaphoreType.DMA,
            pltpu.SemaphoreType.DMA,
        ],
    )


NB = 10
RB = N // NB


def _dense1_body(eps_ref, x_ref, agg_ref, w1_ref, b1_ref,
                 h1_ref, sum_ref, sq_ref):
    x = x_ref[...]
    h = (1.0 + eps_ref[0]) * x + agg_ref[0] + agg_ref[1]
    h1 = jnp.dot(h, w1_ref[...], preferred_element_type=jnp.float32) \
        + b1_ref[...]
    h1_ref[...] = h1
    sum_ref[0] = jnp.sum(h1, axis=0, keepdims=True)
    sq_ref[0] = jnp.sum(h1 * h1, axis=0, keepdims=True)


def _dense2_body(first, last, *refs):
    if last:
        (x_ref, h1_ref, sum_ref, sq_ref, g_ref, bt_ref, w2_ref, b2_ref,
         o_ref) = refs
    else:
        (x_ref, h1_ref, sum_ref, sq_ref, g_ref, bt_ref, w2_ref, b2_ref,
         bn_ref, o_ref, y_ref) = refs
    h1 = h1_ref[...]
    mean = jnp.sum(sum_ref[...], axis=0) * (1.0 / N)
    ex2 = jnp.sum(sq_ref[...], axis=0) * (1.0 / N)
    var = ex2 - mean * mean
    hn = (h1 - mean) * jax.lax.rsqrt(var + 1e-5) * g_ref[...] + bt_ref[...]
    h2 = jnp.maximum(hn, 0.0)
    h3 = jnp.dot(h2, w2_ref[...], preferred_element_type=jnp.float32) \
        + b2_ref[...]
    r = jnp.maximum(h3, 0.0)
    o = r if first else x_ref[...] + r
    o_ref[...] = o
    if not last:
        y_ref[...] = o + bn_ref[...]


def _prep_body(x_ref, b_ref, y_ref):
    y_ref[...] = x_ref[...] + b_ref[...]


def _prep_call(x, b):
    return pl.pallas_call(
        _prep_body,
        grid=(NB,),
        in_specs=[pl.BlockSpec((RB, D), _row_block),
                  pl.BlockSpec((1, D), lambda i: (0, 0))],
        out_specs=pl.BlockSpec((RB, D), _row_block),
        out_shape=jax.ShapeDtypeStruct((N, D), jnp.float32),
    )(x, b)


def _row_block(i):
    return (i, 0)


def _dense1_call(eps, x, agg, w1, b1):
    return pl.pallas_call(
        _dense1_body,
        grid=(NB,),
        in_specs=[
            pl.BlockSpec(memory_space=pltpu.SMEM),
            pl.BlockSpec((RB, D), _row_block),
            pl.BlockSpec((2, RB, D), lambda i: (0, i, 0)),
            pl.BlockSpec((D, D), lambda i: (0, 0)),
            pl.BlockSpec((1, D), lambda i: (0, 0)),
        ],
        out_specs=[
            pl.BlockSpec((RB, D), _row_block),
            pl.BlockSpec((1, 1, D), lambda i: (i, 0, 0)),
            pl.BlockSpec((1, 1, D), lambda i: (i, 0, 0)),
        ],
        out_shape=[
            jax.ShapeDtypeStruct((N, D), jnp.float32),
            jax.ShapeDtypeStruct((NB, 1, D), jnp.float32),
            jax.ShapeDtypeStruct((NB, 1, D), jnp.float32),
        ],
    )(eps, x, agg, w1, b1)


def _dense2_call(first, last, x, h1, sums, sqs, gamma, beta, w2, b2,
                 bn=None):
    vec = lambda i: (0, 0)
    in_specs = [
        pl.BlockSpec((RB, D), _row_block),
        pl.BlockSpec((RB, D), _row_block),
        pl.BlockSpec((NB, 1, D), lambda i: (0, 0, 0)),
        pl.BlockSpec((NB, 1, D), lambda i: (0, 0, 0)),
        pl.BlockSpec((1, D), vec),
        pl.BlockSpec((1, D), vec),
        pl.BlockSpec((D, D), vec),
        pl.BlockSpec((1, D), vec),
    ]
    args = [x, h1, sums, sqs, gamma, beta, w2, b2]
    out_specs = pl.BlockSpec((RB, D), _row_block)
    out_shape = jax.ShapeDtypeStruct((N, D), jnp.float32)
    if not last:
        in_specs.append(pl.BlockSpec((1, D), vec))
        args.append(bn)
        out_specs = [out_specs, pl.BlockSpec((RB, D), _row_block)]
        out_shape = [out_shape, jax.ShapeDtypeStruct((N, D), jnp.float32)]
    return pl.pallas_call(
        functools.partial(_dense2_body, first, last),
        grid=(NB,),
        in_specs=in_specs,
        out_specs=out_specs,
        out_shape=out_shape,
    )(*args)


def kernel(x, edge_index, edge_attr, params):
    x = x.astype(jnp.float32)
    src = edge_index[0].astype(jnp.int32).reshape(N_CHUNKS, CH)
    dst = edge_index[1].astype(jnp.int32).reshape(N_CHUNKS, CH)
    a2 = edge_attr.astype(jnp.float32).reshape(N_CHUNKS, CH)
    nl = len(params)
    y = _prep_call(x, params[0]["be"].reshape(1, D))
    for li, p in enumerate(params):
        web = p["We"].reshape(1, D)
        agg = _edge_call()(y, src, dst, a2, web)
        eps = p["eps"].reshape(1)
        h1, sums, sqs = _dense1_call(eps, x, agg, p["W1"],
                                     p["b1"].reshape(1, D))
        last = li == nl - 1
        if last:
            x = _dense2_call(li == 0, True, x, h1, sums, sqs,
                             p["gamma"].reshape(1, D),
                             p["beta"].reshape(1, D),
                             p["W2"], p["b2"].reshape(1, D))
        else:
            x, y = _dense2_call(li == 0, False, x, h1, sums, sqs,
                                p["gamma"].reshape(1, D),
                                p["beta"].reshape(1, D),
                                p["W2"], p["b2"].reshape(1, D),
                                params[li + 1]["be"].reshape(1, D))
    return x


# speedup vs baseline: 1.1009x; 1.1009x over previous
import functools

import jax
import jax.numpy as jnp
from jax import lax
from jax.experimental import pallas as pl
from jax.experimental.pallas import tpu as pltpu
from jax.experimental.pallas import tpu_sc as plsc

N = 10000
E = 320000
D = 128
NC, NS = 2, 16
NW = NC * NS
CH = 128
N_CHUNKS = E // CH
NT_BASE = N_CHUNKS // NW
NT_XTRA = N_CHUNKS % NW
RPT = 624
REM_BASE = NS * RPT
REM = N - REM_BASE


def _edge_body(x_hbm, src_hbm, dst_hbm, a_hbm, web_hbm, out_hbm,
               sb0, sb1, sb2, db0, db1, db2, av0, av1, av2,
               rows0, rows1, rows2,
               web_v, agg_sh, sem_g, sem_s, sem_i, sem_c, sem_d):
    sb_s = (sb0, sb1, sb2)
    db_s = (db0, db1, db2)
    av_s = (av0, av1, av2)
    rows_s = (rows0, rows1, rows2)
    c = lax.axis_index("c")
    s = lax.axis_index("s")
    w = s * NC + c

    def _zrow(i, _):
        for j in range(8):
            rows0[i, pl.ds(j * 16, 16)] = jnp.zeros((16,), jnp.float32)
        return 0
    lax.fori_loop(0, CH, _zrow, 0)
    zsrc = rows0
    base_r = s * RPT
    for kk in range(RPT // CH):
        pltpu.sync_copy(zsrc, agg_sh.at[pl.ds(base_r + kk * CH, CH)])
    rem = RPT % CH
    pltpu.sync_copy(zsrc.at[pl.ds(0, rem)],
                    agg_sh.at[pl.ds(base_r + (RPT // CH) * CH, rem)])

    @pl.when(s == NS - 1)
    def _zero_tail():
        pltpu.sync_copy(zsrc.at[pl.ds(0, REM)],
                        agg_sh.at[pl.ds(REM_BASE, REM)])

    pltpu.sync_copy(web_hbm, web_v)
    plsc.subcore_barrier()

    wej = [web_v[0, pl.ds(j * 16, 16)] for j in range(8)]

    has_extra = w < NT_XTRA

    def _gather(slot):
        return pltpu.make_async_copy(
            x_hbm.at[sb_s[slot]], rows_s[slot], sem_g)

    def _scatter(slot):
        return pltpu.make_async_copy(rows_s[slot],
                                     agg_sh.at[db_s[slot]], sem_s)

    def _scopy(t, slot):
        return pltpu.make_async_copy(src_hbm.at[w + t * NW], sb_s[slot],
                                     sem_c)

    def _dcopy(t, slot):
        return pltpu.make_async_copy(dst_hbm.at[w + t * NW], db_s[slot],
                                     sem_d)

    def _acopy(t, slot):
        return pltpu.make_async_copy(a_hbm.at[w + t * NW], av_s[slot],
                                     sem_i)

    def _compute(t, slot):
        _acopy(t, slot).wait()
        _gather(slot).wait()
        rv = rows_s[slot]
        av = av_s[slot]

        @plsc.parallel_loop(0, CH // 16, unroll=2)
        def _edges(i16):
            a16 = av[pl.ds(i16 * 16, 16)]
            for u in range(16):
                i = i16 * 16 + u
                a = a16[u]
                for jj in range(8):
                    sl = pl.ds(jj * 16, 16)
                    rv[i, sl] = jnp.maximum(
                        rv[i, sl] + a * wej[jj], 0.0)

    def _scatter_start(slot):
        pltpu.async_copy(rows_s[slot], agg_sh.at[db_s[slot]], sem_s,
                         add=True)

    _scopy(0, 0).start()
    _scopy(0, 0).wait()
    _gather(0).start()
    _scopy(1, 1).start()
    _dcopy(0, 0).start()
    _acopy(0, 0).start()
    _acopy(1, 1).start()
    plsc.subcore_barrier()

    def _chunk(t, slot, waitprev_cond, prefetch_cond, acopy_cond):
        @pl.when(waitprev_cond)
        def _wp():
            _scatter((slot + 1) % 3).wait()

        @pl.when(prefetch_cond)
        def _pf():
            _dcopy(t + 1, (slot + 1) % 3).start()
            _scopy(t + 1, (slot + 1) % 3).wait()
            _gather((slot + 1) % 3).start()

        @pl.when(acopy_cond)
        def _ac():
            _scopy(t + 2, (slot + 2) % 3).start()
            _acopy(t + 2, (slot + 2) % 3).start()
        _compute(t, slot)
        _dcopy(t, slot).wait()
        _scatter_start(slot)

    def _trip(p, _):
        t0 = p * 3
        last_trip = p == NT_BASE // 3 - 1
        _chunk(t0, 0, p > 0, True, True)
        _chunk(t0 + 1, 1, p > 0, True, (~last_trip) | has_extra)
        _chunk(t0 + 2, 2, True, (~last_trip) | has_extra, ~last_trip)
        return 0
    lax.fori_loop(0, NT_BASE // 3, _trip, 0)

    _scatter(1).wait()

    @pl.when(has_extra)
    def _extra():
        _compute(NT_BASE, 0)
        _dcopy(NT_BASE, 0).wait()
        _scatter_start(0)

    _scatter(2).wait()

    @pl.when(has_extra)
    def _wait_extra():
        _scatter(0).wait()

    plsc.subcore_barrier()
    pltpu.sync_copy(agg_sh.at[pl.ds(s * RPT, RPT)],
                    out_hbm.at[c, pl.ds(s * RPT, RPT)])

    @pl.when(s == NS - 1)
    def _dump_tail():
        pltpu.sync_copy(agg_sh.at[pl.ds(REM_BASE, REM)],
                        out_hbm.at[c, pl.ds(REM_BASE, REM)])


@functools.cache
def _edge_call():
    return pl.kernel(
        _edge_body,
        out_type=jax.ShapeDtypeStruct((NC, N, D), jnp.float32),
        mesh=plsc.VectorSubcoreMesh(core_axis_name="c", subcore_axis_name="s",
                                    num_cores=NC, num_subcores=NS),
        scratch_types=[
            pltpu.VMEM((CH,), jnp.int32),
            pltpu.VMEM((CH,), jnp.int32),
            pltpu.VMEM((CH,), jnp.int32),
            pltpu.VMEM((CH,), jnp.int32),
            pltpu.VMEM((CH,), jnp.int32),
            pltpu.VMEM((CH,), jnp.int32),
            pltpu.VMEM((CH,), jnp.float32),
            pltpu.VMEM((CH,), jnp.float32),
            pltpu.VMEM((CH,), jnp.float32),
            pltpu.VMEM((CH, D), jnp.float32),
            pltpu.VMEM((CH, D), jnp.float32),
            pltpu.VMEM((CH, D), jnp.float32),
            pltpu.VMEM((1, D), jnp.float32),
            pltpu.VMEM_SHARED((N, D), jnp.float32),
            pltpu.SemaphoreType.DMA,
            pltpu.SemaphoreType.DMA,
            pltpu.SemaphoreType.DMA,
            pltpu.SemaphoreType.DMA,
            pltpu.SemaphoreType.DMA,
        ],
    )


NB = 10
RB = N // NB


def _dense1_body(eps_ref, x_ref, agg_ref, w1_ref, b1_ref,
                 h1_ref, sum_ref, sq_ref):
    x = x_ref[...]
    h = (1.0 + eps_ref[0]) * x + agg_ref[0] + agg_ref[1]
    h1 = jnp.dot(h, w1_ref[...], preferred_element_type=jnp.float32) \
        + b1_ref[...]
    h1_ref[...] = h1
    sum_ref[0] = jnp.sum(h1, axis=0, keepdims=True)
    sq_ref[0] = jnp.sum(h1 * h1, axis=0, keepdims=True)


def _dense2_body(first, last, *refs):
    if last:
        (x_ref, h1_ref, sum_ref, sq_ref, g_ref, bt_ref, w2_ref, b2_ref,
         o_ref) = refs
    else:
        (x_ref, h1_ref, sum_ref, sq_ref, g_ref, bt_ref, w2_ref, b2_ref,
         bn_ref, o_ref, y_ref) = refs
    h1 = h1_ref[...]
    mean = jnp.sum(sum_ref[...], axis=0) * (1.0 / N)
    ex2 = jnp.sum(sq_ref[...], axis=0) * (1.0 / N)
    var = ex2 - mean * mean
    hn = (h1 - mean) * jax.lax.rsqrt(var + 1e-5) * g_ref[...] + bt_ref[...]
    h2 = jnp.maximum(hn, 0.0)
    h3 = jnp.dot(h2, w2_ref[...], preferred_element_type=jnp.float32) \
        + b2_ref[...]
    r = jnp.maximum(h3, 0.0)
    o = r if first else x_ref[...] + r
    o_ref[...] = o
    if not last:
        y_ref[...] = o + bn_ref[...]


def _prep_body(x_ref, b_ref, y_ref):
    y_ref[...] = x_ref[...] + b_ref[...]


def _prep_call(x, b):
    return pl.pallas_call(
        _prep_body,
        grid=(NB,),
        in_specs=[pl.BlockSpec((RB, D), _row_block),
                  pl.BlockSpec((1, D), lambda i: (0, 0))],
        out_specs=pl.BlockSpec((RB, D), _row_block),
        out_shape=jax.ShapeDtypeStruct((N, D), jnp.float32),
    )(x, b)


def _row_block(i):
    return (i, 0)


def _dense1_call(eps, x, agg, w1, b1):
    return pl.pallas_call(
        _dense1_body,
        grid=(NB,),
        in_specs=[
            pl.BlockSpec(memory_space=pltpu.SMEM),
            pl.BlockSpec((RB, D), _row_block),
            pl.BlockSpec((2, RB, D), lambda i: (0, i, 0)),
            pl.BlockSpec((D, D), lambda i: (0, 0)),
            pl.BlockSpec((1, D), lambda i: (0, 0)),
        ],
        out_specs=[
            pl.BlockSpec((RB, D), _row_block),
            pl.BlockSpec((1, 1, D), lambda i: (i, 0, 0)),
            pl.BlockSpec((1, 1, D), lambda i: (i, 0, 0)),
        ],
        out_shape=[
            jax.ShapeDtypeStruct((N, D), jnp.float32),
            jax.ShapeDtypeStruct((NB, 1, D), jnp.float32),
            jax.ShapeDtypeStruct((NB, 1, D), jnp.float32),
        ],
    )(eps, x, agg, w1, b1)


def _dense2_call(first, last, x, h1, sums, sqs, gamma, beta, w2, b2,
                 bn=None):
    vec = lambda i: (0, 0)
    in_specs = [
        pl.BlockSpec((RB, D), _row_block),
        pl.BlockSpec((RB, D), _row_block),
        pl.BlockSpec((NB, 1, D), lambda i: (0, 0, 0)),
        pl.BlockSpec((NB, 1, D), lambda i: (0, 0, 0)),
        pl.BlockSpec((1, D), vec),
        pl.BlockSpec((1, D), vec),
        pl.BlockSpec((D, D), vec),
        pl.BlockSpec((1, D), vec),
    ]
    args = [x, h1, sums, sqs, gamma, beta, w2, b2]
    out_specs = pl.BlockSpec((RB, D), _row_block)
    out_shape = jax.ShapeDtypeStruct((N, D), jnp.float32)
    if not last:
        in_specs.append(pl.BlockSpec((1, D), vec))
        args.append(bn)
        out_specs = [out_specs, pl.BlockSpec((RB, D), _row_block)]
        out_shape = [out_shape, jax.ShapeDtypeStruct((N, D), jnp.float32)]
    return pl.pallas_call(
        functools.partial(_dense2_body, first, last),
        grid=(NB,),
        in_specs=in_specs,
        out_specs=out_specs,
        out_shape=out_shape,
    )(*args)


def kernel(x, edge_index, edge_attr, params):
    x = x.astype(jnp.float32)
    src = edge_index[0].astype(jnp.int32).reshape(N_CHUNKS, CH)
    dst = edge_index[1].astype(jnp.int32).reshape(N_CHUNKS, CH)
    a2 = edge_attr.astype(jnp.float32).reshape(N_CHUNKS, CH)
    nl = len(params)
    y = _prep_call(x, params[0]["be"].reshape(1, D))
    for li, p in enumerate(params):
        web = p["We"].reshape(1, D)
        agg = _edge_call()(y, src, dst, a2, web)
        eps = p["eps"].reshape(1)
        h1, sums, sqs = _dense1_call(eps, x, agg, p["W1"],
                                     p["b1"].reshape(1, D))
        last = li == nl - 1
        if last:
            x = _dense2_call(li == 0, True, x, h1, sums, sqs,
                             p["gamma"].reshape(1, D),
                             p["beta"].reshape(1, D),
                             p["W2"], p["b2"].reshape(1, D))
        else:
            x, y = _dense2_call(li == 0, False, x, h1, sums, sqs,
                                p["gamma"].reshape(1, D),
                                p["beta"].reshape(1, D),
                                p["W2"], p["b2"].reshape(1, D),
                                params[li + 1]["be"].reshape(1, D))
    return x
